# Initial kernel scaffold; baseline (speedup 1.0000x reference)
import jax
import jax.numpy as jnp
from jax.experimental import pallas as pl


def kernel(W_poi, att_W, att_b, v_attention, mask, parent_ids, children):
    raise NotImplementedError("write your pallas kernel here")


# speedup vs baseline: 2.5505x; 2.5505x over previous
import functools

import jax
import jax.numpy as jnp
from jax import lax
from jax.experimental import pallas as pl
from jax.experimental.pallas import tpu as pltpu
from jax.experimental.pallas import tpu_sc as plsc

N_NODES = 10000
P = 5000
C = 64
D = 128
ATT = 64

_NC, _NS = 2, 16
_NW = _NC * _NS
_TOTAL = P * C
_PER_W = _TOTAL // _NW
_CHUNK = 80
_NITER = _PER_W // _CHUNK


def _sc_gather_body(table_hbm, idx_hbm, out_hbm, idx_v, rows_v, sem):
    wid = lax.axis_index("s") * _NC + lax.axis_index("c")

    def body(j, _):
        base = pl.multiple_of(wid * _PER_W + j * _CHUNK, 8)
        pltpu.sync_copy(idx_hbm.at[pl.ds(base, _CHUNK)], idx_v)
        pltpu.async_copy(table_hbm.at[idx_v], rows_v, sem).wait()
        pltpu.sync_copy(rows_v, out_hbm.at[pl.ds(base, _CHUNK)])
        return ()

    lax.fori_loop(0, _NITER, body, ())


def _sc_gather(table, idx_flat):
    mesh = plsc.VectorSubcoreMesh(core_axis_name="c", subcore_axis_name="s")
    f = functools.partial(
        pl.kernel,
        mesh=mesh,
        out_type=jax.ShapeDtypeStruct((_TOTAL, D), jnp.float32),
        scratch_types=[
            pltpu.VMEM((_CHUNK,), jnp.int32),
            pltpu.VMEM((_CHUNK, D), jnp.float32),
            pltpu.SemaphoreType.DMA,
        ],
    )(_sc_gather_body)
    return f(table, idx_flat)


_BP = 200
_NBLK = P // _BP
_NGRID = N_NODES // _BP


def _tc_dense_body(wpoi_ref, gath_ref, attw_ref, b_ref, v_ref, mask_ref,
                   out_ref, s_ref):
    i = pl.program_id(0)

    @pl.when(i < _NBLK)
    def _compute():
        wp = wpoi_ref[...]
        top = attw_ref[:D, :]
        bot = attw_ref[D:, :]
        pp = jnp.dot(wp, top, preferred_element_type=jnp.float32)
        pp = pp + b_ref[...]
        v = v_ref[...]
        for c in range(C):
            xc = gath_ref[c]
            cp = jnp.dot(xc, bot, preferred_element_type=jnp.float32)
            pre = jnp.tanh(pp + cp)
            s_ref[:, c:c + 1] = jnp.sum(pre * v, axis=1, keepdims=True)
        att = jax.nn.softmax(s_ref[...] + mask_ref[...], axis=1)
        acc = gath_ref[0] * att[:, 0:1]
        for c in range(1, C):
            acc = acc + gath_ref[c] * att[:, c:c + 1]
        out_ref[...] = acc

    @pl.when(i >= _NBLK)
    def _copy():
        out_ref[...] = wpoi_ref[...]


def _tc_dense(W_poi, gathered, att_W, att_b, v_attention, mask):
    clamp = lambda i: (jnp.minimum(i, _NBLK - 1),)
    return pl.pallas_call(
        _tc_dense_body,
        grid=(_NGRID,),
        in_specs=[
            pl.BlockSpec((_BP, D), lambda i: (i, 0)),
            pl.BlockSpec((C, _BP, D), lambda i: (0, jnp.minimum(i, _NBLK - 1), 0)),
            pl.BlockSpec((2 * D, ATT), lambda i: (0, 0)),
            pl.BlockSpec((1, ATT), lambda i: (0, 0)),
            pl.BlockSpec((1, ATT), lambda i: (0, 0)),
            pl.BlockSpec((_BP, C), lambda i: (jnp.minimum(i, _NBLK - 1), 0)),
        ],
        out_specs=pl.BlockSpec((_BP, D), lambda i: (i, 0)),
        out_shape=jax.ShapeDtypeStruct((N_NODES, D), jnp.float32),
        scratch_shapes=[pltpu.VMEM((_BP, C), jnp.float32)],
    )(W_poi, gathered, att_W, att_b, v_attention, mask)


def kernel(W_poi, att_W, att_b, v_attention, mask, parent_ids, children):
    idx_flat = jnp.transpose(children).reshape(-1).astype(jnp.int32)
    gathered = _sc_gather(W_poi, idx_flat)
    gathered = gathered.reshape(C, P, D)
    return _tc_dense(W_poi, gathered, att_W,
                     att_b.reshape(1, ATT), v_attention.reshape(1, ATT), mask)


# speedup vs baseline: 3.9148x; 1.5349x over previous
import functools

import jax
import jax.numpy as jnp
from jax import lax
from jax.experimental import pallas as pl
from jax.experimental.pallas import tpu as pltpu
from jax.experimental.pallas import tpu_sc as plsc

N_NODES = 10000
P = 5000
C = 64
D = 128
ATT = 64

_NC, _NS = 2, 16
_NW = _NC * _NS
_TOTAL = P * C
_PER_W = _TOTAL // _NW
_CHUNK = 80
_NITER = _PER_W // _CHUNK


_NBUF = 5


def _sc_gather_body(table_hbm, idx_hbm, out_hbm, idx_v, *bufs):
    rows = bufs[:_NBUF]
    gsems = bufs[_NBUF:2 * _NBUF]
    wsems = bufs[2 * _NBUF:]
    wid = lax.axis_index("s") * _NC + lax.axis_index("c")
    base0 = pl.multiple_of(wid * _PER_W, 8)
    pltpu.sync_copy(idx_hbm.at[pl.ds(base0, _PER_W)], idx_v)

    def _start_gather(g, k):
        off = pl.multiple_of(g * _CHUNK, 8)
        pltpu.async_copy(table_hbm.at[idx_v.at[pl.ds(off, _CHUNK)]],
                         rows[k], gsems[k])

    def _finish(g, k):
        pltpu.make_async_copy(table_hbm.at[idx_v.at[pl.ds(0, _CHUNK)]],
                              rows[k], gsems[k]).wait()
        out_off = pl.multiple_of(base0 + g * _CHUNK, 8)
        pltpu.async_copy(rows[k], out_hbm.at[pl.ds(out_off, _CHUNK)],
                         wsems[k])

    def _wait_write(k):
        pltpu.make_async_copy(rows[k],
                              out_hbm.at[pl.ds(base0, _CHUNK)],
                              wsems[k]).wait()

    for k in range(_NBUF):
        _start_gather(k, k)

    def body(jj, _):
        g0 = _NBUF * jj
        for k in range(_NBUF):
            _finish(g0 + k, k)
        for k in range(_NBUF):
            _wait_write(k)
            _start_gather(g0 + _NBUF + k, k)
        return ()

    lax.fori_loop(0, _NITER // _NBUF - 1, body, ())
    g0 = _NITER - _NBUF
    for k in range(_NBUF):
        _finish(g0 + k, k)
    for k in range(_NBUF):
        _wait_write(k)


def _sc_gather(table, idx_flat):
    mesh = plsc.VectorSubcoreMesh(core_axis_name="c", subcore_axis_name="s")
    f = functools.partial(
        pl.kernel,
        mesh=mesh,
        out_type=jax.ShapeDtypeStruct((_TOTAL, D), jnp.float32),
        scratch_types=(
            [pltpu.VMEM((_PER_W,), jnp.int32)]
            + [pltpu.VMEM((_CHUNK, D), jnp.float32) for _ in range(_NBUF)]
            + [pltpu.SemaphoreType.DMA for _ in range(2 * _NBUF)]
        ),
    )(_sc_gather_body)
    return f(table, idx_flat)


_BP = 200
_NBLK = P // _BP
_NGRID = N_NODES // _BP


def _tc_dense_body(wpoi_ref, gath_ref, attw_ref, b_ref, v_ref, mask_ref,
                   out_ref, s_ref):
    i = pl.program_id(0)

    @pl.when(i < _NBLK)
    def _compute():
        wp = wpoi_ref[...]
        top = attw_ref[:D, :]
        bot = attw_ref[D:, :]
        pp = jnp.dot(wp, top, preferred_element_type=jnp.float32)
        pp = pp + b_ref[...]
        xall = gath_ref[...].reshape(C * _BP, D)
        cp = jnp.dot(xall, bot, preferred_element_type=jnp.float32)
        pptile = jnp.broadcast_to(pp[None], (C, _BP, ATT)).reshape(C * _BP, ATT)
        pre = jnp.tanh(cp + pptile)
        sflat = jnp.dot(pre, v_ref[...],
                        preferred_element_type=jnp.float32)
        for c in range(C):
            s_ref[:, c:c + 1] = sflat[c * _BP:(c + 1) * _BP]
        att = jax.nn.softmax(s_ref[...] + mask_ref[...], axis=1)
        acc = gath_ref[0] * att[:, 0:1]
        for c in range(1, C):
            acc = acc + gath_ref[c] * att[:, c:c + 1]
        out_ref[...] = acc

    @pl.when(i >= _NBLK)
    def _copy():
        out_ref[...] = wpoi_ref[...]


def _tc_dense(W_poi, gathered, att_W, att_b, v_attention, mask):
    clamp = lambda i: (jnp.minimum(i, _NBLK - 1),)
    return pl.pallas_call(
        _tc_dense_body,
        grid=(_NGRID,),
        in_specs=[
            pl.BlockSpec((_BP, D), lambda i: (i, 0)),
            pl.BlockSpec((C, _BP, D), lambda i: (0, jnp.minimum(i, _NBLK - 1), 0)),
            pl.BlockSpec((2 * D, ATT), lambda i: (0, 0)),
            pl.BlockSpec((1, ATT), lambda i: (0, 0)),
            pl.BlockSpec((ATT, 1), lambda i: (0, 0)),
            pl.BlockSpec((_BP, C), lambda i: (jnp.minimum(i, _NBLK - 1), 0)),
        ],
        out_specs=pl.BlockSpec((_BP, D), lambda i: (i, 0)),
        out_shape=jax.ShapeDtypeStruct((N_NODES, D), jnp.float32),
        scratch_shapes=[pltpu.VMEM((_BP, C), jnp.float32)],
    )(W_poi, gathered, att_W, att_b, v_attention, mask)


def kernel(W_poi, att_W, att_b, v_attention, mask, parent_ids, children):
    idx_flat = jnp.transpose(children).reshape(-1).astype(jnp.int32)
    gathered = _sc_gather(W_poi, idx_flat)
    gathered = gathered.reshape(C, P, D)
    return _tc_dense(W_poi, gathered, att_W,
                     att_b.reshape(1, ATT), v_attention.reshape(ATT, 1), mask)


# speedup vs baseline: 4.3579x; 1.1132x over previous
import functools

import jax
import jax.numpy as jnp
from jax import lax
from jax.experimental import pallas as pl
from jax.experimental.pallas import tpu as pltpu
from jax.experimental.pallas import tpu_sc as plsc

N_NODES = 10000
P = 5000
C = 64
D = 128
ATT = 64

_NC, _NS = 2, 16
_NW = _NC * _NS
_CHUNK = 80
_NBUF = 5
_PA = 3000
_PB = P - _PA


def _sc_gather_body(per_w, table_hbm, idx_hbm, out_hbm, idx_v, *bufs):
    niter = per_w // _CHUNK
    rows = bufs[:_NBUF]
    gsems = bufs[_NBUF:2 * _NBUF]
    wsems = bufs[2 * _NBUF:]
    wid = lax.axis_index("s") * _NC + lax.axis_index("c")
    base0 = pl.multiple_of(wid * per_w, 8)
    pltpu.sync_copy(idx_hbm.at[pl.ds(base0, per_w)], idx_v)

    def _start_gather(g, k):
        off = pl.multiple_of(g * _CHUNK, 8)
        pltpu.async_copy(table_hbm.at[idx_v.at[pl.ds(off, _CHUNK)]],
                         rows[k], gsems[k])

    def _finish(g, k):
        pltpu.make_async_copy(table_hbm.at[idx_v.at[pl.ds(0, _CHUNK)]],
                              rows[k], gsems[k]).wait()
        out_off = pl.multiple_of(base0 + g * _CHUNK, 8)
        pltpu.async_copy(rows[k], out_hbm.at[pl.ds(out_off, _CHUNK)],
                         wsems[k])

    def _wait_write(k):
        pltpu.make_async_copy(rows[k],
                              out_hbm.at[pl.ds(base0, _CHUNK)],
                              wsems[k]).wait()

    for k in range(_NBUF):
        _start_gather(k, k)

    def body(jj, _):
        g0 = _NBUF * jj
        for k in range(_NBUF):
            _finish(g0 + k, k)
        for k in range(_NBUF):
            _wait_write(k)
            _start_gather(g0 + _NBUF + k, k)
        return ()

    lax.fori_loop(0, niter // _NBUF - 1, body, ())
    g0 = niter - _NBUF
    for k in range(_NBUF):
        _finish(g0 + k, k)
    for k in range(_NBUF):
        _wait_write(k)


def _sc_gather(table, idx_flat, n_parents):
    total = n_parents * C
    per_w = total // _NW
    mesh = plsc.VectorSubcoreMesh(core_axis_name="c", subcore_axis_name="s")
    f = functools.partial(
        pl.kernel,
        mesh=mesh,
        out_type=jax.ShapeDtypeStruct((total, D), jnp.float32),
        scratch_types=(
            [pltpu.VMEM((per_w,), jnp.int32)]
            + [pltpu.VMEM((_CHUNK, D), jnp.float32) for _ in range(_NBUF)]
            + [pltpu.SemaphoreType.DMA for _ in range(2 * _NBUF)]
        ),
    )(functools.partial(_sc_gather_body, per_w))
    return f(table, idx_flat)


_BP = 200


def _tc_dense_body(wpoi_ref, gath_ref, attw_ref, b_ref, v_ref, mask_ref,
                   out_ref, s_ref):
    wp = wpoi_ref[...]
    top = attw_ref[:D, :]
    bot = attw_ref[D:, :]
    pp = jnp.dot(wp, top, preferred_element_type=jnp.float32)
    pp = pp + b_ref[...]
    xf = gath_ref[...]
    xall = xf.reshape(C * _BP, D)
    cp = jnp.dot(xall, bot, preferred_element_type=jnp.float32)
    pptile = jnp.broadcast_to(pp[None], (C, _BP, ATT)).reshape(C * _BP, ATT)
    pre = jnp.tanh(cp + pptile)
    sflat = jnp.dot(pre, v_ref[...],
                    preferred_element_type=jnp.float32)
    for c in range(C):
        s_ref[:, c:c + 1] = sflat[c * _BP:(c + 1) * _BP]
    att = jax.nn.softmax(s_ref[...] + mask_ref[...], axis=1)
    acc = xf[0] * att[:, 0:1]
    for c in range(1, C):
        acc = acc + xf[c] * att[:, c:c + 1]
    out_ref[...] = acc


def _tc_dense(W_poi, gathered, att_W, att_b, v_attention, mask, p0, np_):
    nblk = np_ // _BP
    blk0 = p0 // _BP
    return pl.pallas_call(
        _tc_dense_body,
        grid=(nblk,),
        in_specs=[
            pl.BlockSpec((_BP, D), lambda i: (i + blk0, 0)),
            pl.BlockSpec((C, _BP, D), lambda i: (0, i, 0)),
            pl.BlockSpec((2 * D, ATT), lambda i: (0, 0)),
            pl.BlockSpec((1, ATT), lambda i: (0, 0)),
            pl.BlockSpec((ATT, 1), lambda i: (0, 0)),
            pl.BlockSpec((_BP, C), lambda i: (i + blk0, 0)),
        ],
        out_specs=pl.BlockSpec((_BP, D), lambda i: (i, 0)),
        out_shape=jax.ShapeDtypeStruct((np_, D), jnp.float32),
        scratch_shapes=[pltpu.VMEM((_BP, C), jnp.float32)],
    )(W_poi, gathered, att_W, att_b, v_attention, mask)


_BC = 1000


def _tc_copy_body(w_ref, out_ref):
    out_ref[...] = w_ref[...]


def _tc_copy(W_poi):
    nblk = (N_NODES - P) // _BC
    off = P // _BC
    return pl.pallas_call(
        _tc_copy_body,
        grid=(nblk,),
        in_specs=[pl.BlockSpec((_BC, D), lambda i: (i + off, 0))],
        out_specs=pl.BlockSpec((_BC, D), lambda i: (i, 0)),
        out_shape=jax.ShapeDtypeStruct((N_NODES - P, D), jnp.float32),
    )(W_poi)


def kernel(W_poi, att_W, att_b, v_attention, mask, parent_ids, children):
    ch = children.astype(jnp.int32)
    idx_a = jnp.transpose(ch[:_PA]).reshape(-1)
    idx_b = jnp.transpose(ch[_PA:]).reshape(-1)
    b2 = att_b.reshape(1, ATT)
    v2 = v_attention.reshape(ATT, 1)

    gath_a = _sc_gather(W_poi, idx_a, _PA).reshape(C, _PA, D)
    gath_b = _sc_gather(W_poi, idx_b, _PB).reshape(C, _PB, D)
    out_a = _tc_dense(W_poi, gath_a, att_W, b2, v2, mask, 0, _PA)
    out_b = _tc_dense(W_poi, gath_b, att_W, b2, v2, mask, _PA, _PB)
    tail = _tc_copy(W_poi)
    return jnp.concatenate([out_a, out_b, tail], axis=0)
